# Initial kernel scaffold; baseline (speedup 1.0000x reference)
import jax
import jax.numpy as jnp
from jax.experimental import pallas as pl


def kernel(lm_embed, enc_embed, u, idNbests):
    raise NotImplementedError("write your pallas kernel here")


# speedup vs baseline: 24.4422x; 24.4422x over previous
import jax
import jax.numpy as jnp
from jax import lax
from jax.experimental import pallas as pl
from jax.experimental.pallas import tpu as pltpu
from jax.experimental.pallas import tpu_sc as plsc

BB = 4096
MM = 8
LL = 50
VV = 100000
DD = 64
LAM = 0.2

NC = 2
NS = 16
NW = NC * NS
SEGS = BB * MM
SEG_PER_W = SEGS // NW
CHUNK_SEGS = 2
TOK_PER_CHUNK = CHUNK_SEGS * LL
CHUNKS_PER_W = SEG_PER_W // CHUNK_SEGS
IDX_ROWS = SEGS * LL // TOK_PER_CHUNK

VBLK = VV // 4


def _logits_body(lm_ref, u_ref, logits_ref, lse_ref, acc_ref):
    i = pl.program_id(0)
    lv = jnp.sum(lm_ref[...] * u_ref[...], axis=1)
    logits_ref[...] = lv.reshape(1, 1, VBLK)
    bmax = jnp.max(lv)

    @pl.when(i == 0)
    def _():
        acc_ref[0] = bmax
        acc_ref[1] = jnp.sum(jnp.exp(lv - bmax))

    @pl.when(i > 0)
    def _():
        m_old = acc_ref[0]
        m_new = jnp.maximum(m_old, bmax)
        acc_ref[1] = acc_ref[1] * jnp.exp(m_old - m_new) + jnp.sum(
            jnp.exp(lv - m_new))
        acc_ref[0] = m_new

    @pl.when(i == pl.num_programs(0) - 1)
    def _():
        lse_ref[...] = (acc_ref[0] + jnp.log(acc_ref[1])).reshape(1, 1)


def _run_logits(lm_embed, u):
    return pl.pallas_call(
        _logits_body,
        grid=(VV // VBLK,),
        in_specs=[
            pl.BlockSpec((VBLK, DD), lambda i: (i, 0)),
            pl.BlockSpec((1, DD), lambda i: (0, 0)),
        ],
        out_specs=[
            pl.BlockSpec((1, 1, VBLK), lambda i: (i, 0, 0)),
            pl.BlockSpec((1, 1), lambda i: (0, 0)),
        ],
        out_shape=[
            jax.ShapeDtypeStruct((VV // VBLK, 1, VBLK), jnp.float32),
            jax.ShapeDtypeStruct((1, 1), jnp.float32),
        ],
        scratch_shapes=[pltpu.SMEM((2,), jnp.float32)],
    )(lm_embed, u.reshape(1, DD))


def _sc_body(enc_hbm, logits_hbm, idx_hbm, ysum_hbm, slp_hbm,
             idx_v, rows_v, vals_v, ystage, sstage, gsem, vsem):
    cid = lax.axis_index("c")
    sid = lax.axis_index("s")
    wid = sid * NC + cid
    lane = jnp.arange(16, dtype=jnp.int32)

    def chunk_body(g, carry):
        row_idx = wid * CHUNKS_PER_W + g
        seg0 = row_idx * CHUNK_SEGS
        pltpu.sync_copy(idx_hbm.at[row_idx], idx_v)
        rcp = pltpu.async_copy(enc_hbm.at[idx_v], rows_v, gsem)
        vcp = pltpu.async_copy(logits_hbm.at[idx_v], vals_v, vsem)
        rcp.wait()
        vcp.wait()
        for s in range(CHUNK_SEGS):
            accs = [jnp.zeros((16,), jnp.float32) for _ in range(DD // 16)]
            for l in range(LL):
                r = s * LL + l
                for j in range(DD // 16):
                    accs[j] = accs[j] + rows_v[r, pl.ds(j * 16, 16)]
            for j in range(DD // 16):
                ystage[s, pl.ds(j * 16, 16)] = accs[j]
        pltpu.sync_copy(ystage, ysum_hbm.at[pl.ds(seg0, CHUNK_SEGS)])
        for s in range(CHUNK_SEGS):
            base = s * LL
            t0 = vals_v[pl.ds(base, 16)]
            t1 = vals_v[pl.ds(base + 16, 16)]
            t2 = vals_v[pl.ds(base + 32, 16)]
            t3 = vals_v[pl.ds(base + 34, 16)]
            t3m = jnp.where(lane >= 14, t3, 0.0)
            sstage[s, pl.ds(0, 16)] = t0 + t1 + t2 + t3m
        pltpu.sync_copy(sstage, slp_hbm.at[pl.ds(seg0, CHUNK_SEGS)])
        return carry

    lax.fori_loop(0, CHUNKS_PER_W, chunk_body, 0)


_sc_call = pl.kernel(
    _sc_body,
    out_type=(
        jax.ShapeDtypeStruct((SEGS, DD), jnp.float32),
        jax.ShapeDtypeStruct((SEGS, 16), jnp.float32),
    ),
    mesh=plsc.VectorSubcoreMesh(core_axis_name="c", subcore_axis_name="s"),
    scratch_types=[
        pltpu.VMEM((TOK_PER_CHUNK,), jnp.int32),
        pltpu.VMEM((TOK_PER_CHUNK, DD), jnp.float32),
        pltpu.VMEM((TOK_PER_CHUNK,), jnp.float32),
        pltpu.VMEM((CHUNK_SEGS, DD), jnp.float32),
        pltpu.VMEM((CHUNK_SEGS, 16), jnp.float32),
        pltpu.SemaphoreType.DMA,
        pltpu.SemaphoreType.DMA,
    ],
    compiler_params=pltpu.CompilerParams(use_tc_tiling_on_sc=False),
)


BBLK = 512


def _fin_body(ysum_ref, slp_ref, lse_ref, vss_ref, ul_ref, acc_ref):
    i = pl.program_id(0)
    lse = lse_ref[0, 0]
    slp = jnp.sum(slp_ref[...], axis=2)
    lp = LAM * (slp - LL * lse)
    mx = jnp.max(lp, axis=1, keepdims=True)
    e = jnp.exp(lp - mx)
    attn = e / jnp.sum(e, axis=1, keepdims=True)
    ys = ysum_ref[...]
    vss_ref[...] = jnp.sum(attn[:, :, None] * ys, axis=1) * (1.0 / LL)
    part = jnp.sum(lp * attn)
    prev = jnp.where(i == 0, 0.0, acc_ref[0])
    acc_ref[0] = prev - part

    @pl.when(i == pl.num_programs(0) - 1)
    def _():
        ul_ref[...] = (acc_ref[0] / (LL * SEGS)).reshape(1, 1)


def _run_final(ysum, slp, lse):
    return pl.pallas_call(
        _fin_body,
        grid=(BB // BBLK,),
        in_specs=[
            pl.BlockSpec((BBLK, MM, DD), lambda i: (i, 0, 0)),
            pl.BlockSpec((BBLK, MM, 16), lambda i: (i, 0, 0)),
            pl.BlockSpec((1, 1), lambda i: (0, 0)),
        ],
        out_specs=[
            pl.BlockSpec((BBLK, DD), lambda i: (i, 0)),
            pl.BlockSpec((1, 1), lambda i: (0, 0)),
        ],
        out_shape=[
            jax.ShapeDtypeStruct((BB, DD), jnp.float32),
            jax.ShapeDtypeStruct((1, 1), jnp.float32),
        ],
        scratch_shapes=[pltpu.SMEM((1,), jnp.float32)],
    )(ysum.reshape(BB, MM, DD), slp.reshape(BB, MM, 16), lse)


def kernel(lm_embed, enc_embed, u, idNbests):
    idx2d = idNbests.astype(jnp.int32).reshape(IDX_ROWS, TOK_PER_CHUNK)
    logits2d, lse = _run_logits(lm_embed, u)
    logits_flat = logits2d.reshape(VV)
    ysum, slp2 = _sc_call(enc_embed, logits_flat, idx2d)
    vss, ul = _run_final(ysum, slp2, lse)
    return vss, ul[0, 0]
